# baseline (speedup 1.0000x reference)
import functools

import jax
import jax.numpy as jnp
from jax import lax
from jax.experimental import pallas as pl
from jax.experimental.pallas import tpu as pltpu
from jax.experimental.pallas import tpu_sc as plsc

_VOCAB = 100000
_DIM = 64
_B = 16384
_K = 5

_NC = 2
_NS = 16
_NW = _NC * _NS
_BPW = _B // _NW
_S = 64
_NCHUNK = _BPW // _S
_G = _S // 16


@functools.cache
def _make_sc_logits():
    mesh = plsc.VectorSubcoreMesh(core_axis_name="c", subcore_axis_name="s")

    @functools.partial(
        pl.kernel,
        mesh=mesh,
        compiler_params=pltpu.CompilerParams(
            needs_layout_passes=False, use_tc_tiling_on_sc=False),
        out_type=jax.ShapeDtypeStruct((6 * _B,), jnp.float32),
        scratch_types=[
            pltpu.VMEM((_BPW,), jnp.int32),
            pltpu.VMEM((_BPW,), jnp.int32),
            pltpu.VMEM((_K * _BPW,), jnp.int32),
            pltpu.VMEM((_S, _DIM), jnp.float32),
            pltpu.VMEM((_S, _DIM), jnp.float32),
            pltpu.VMEM((_K * _S, _DIM), jnp.float32),
            pltpu.VMEM((_S, _DIM), jnp.float32),
            pltpu.VMEM((_S, _DIM), jnp.float32),
            pltpu.VMEM((_K * _S, _DIM), jnp.float32),
            pltpu.VMEM((6, _BPW), jnp.float32),
            pltpu.SemaphoreType.DMA,
            pltpu.SemaphoreType.DMA,
            pltpu.SemaphoreType.DMA,
        ],
    )
    def sc_logits(cen_hbm, ctx_hbm, neg_hbm, win_hbm, wout_hbm, out_hbm,
                  idxc, idxx, idxn,
                  crA, xrA, nrA, crB, xrB, nrB,
                  lbuf, semi, semA, semB):
        wid = lax.axis_index("s") * _NC + lax.axis_index("c")
        base = wid * _BPW
        iota = lax.iota(jnp.int32, 16)

        cpi = [pltpu.async_copy(cen_hbm.at[pl.ds(base, _BPW)], idxc, semi),
               pltpu.async_copy(ctx_hbm.at[pl.ds(base, _BPW)], idxx, semi),
               pltpu.async_copy(neg_hbm.at[pl.ds(base * _K, _K * _BPW)],
                                idxn, semi)]
        for cp in cpi:
            cp.wait()

        def gather_bufs(t, cr, xr, nr, sem):
            toff = pl.multiple_of(t * _S, _S)
            return [
                pltpu.async_copy(win_hbm.at[idxc.at[pl.ds(toff, _S)]],
                                 cr, sem),
                pltpu.async_copy(wout_hbm.at[idxx.at[pl.ds(toff, _S)]],
                                 xr, sem),
                pltpu.async_copy(wout_hbm.at[idxn.at[pl.ds(toff * _K,
                                                           _K * _S)]],
                                 nr, sem),
            ]

        def wait_bufs(t, cr, xr, nr, sem):
            toff = pl.multiple_of(t * _S, _S)
            pltpu.make_async_copy(win_hbm.at[idxc.at[pl.ds(toff, _S)]],
                                  cr, sem).wait()
            pltpu.make_async_copy(wout_hbm.at[idxx.at[pl.ds(toff, _S)]],
                                  xr, sem).wait()
            pltpu.make_async_copy(wout_hbm.at[idxn.at[pl.ds(toff * _K,
                                                            _K * _S)]],
                                  nr, sem).wait()

        def compute_chunk(t, cr, xr, nr):
            toff = pl.multiple_of(t * _S, _S)

            def g_body(g, carry):
                s0 = pl.multiple_of(g * 16, 16)
                rows = s0 + iota
                nrows16 = [rows * _K + j for j in range(_K)]
                accs = [jnp.zeros((16,), jnp.float32) for _ in range(6)]
                col = jnp.zeros((16,), jnp.int32)
                one = jnp.ones((16,), jnp.int32)
                for _d in range(_DIM):
                    cv = plsc.load_gather(cr, [rows, col])
                    xv = plsc.load_gather(xr, [rows, col])
                    accs[0] = accs[0] + cv * xv
                    for j in range(_K):
                        nv = plsc.load_gather(nr, [nrows16[j], col])
                        accs[j + 1] = accs[j + 1] + cv * nv
                    col = col + one
                for j in range(6):
                    lbuf[j, pl.ds(toff + s0, 16)] = accs[j]
                return carry

            lax.fori_loop(0, _G, g_body, 0)

        gather_bufs(0, crA, xrA, nrA, semA)

        def pair_body(pr, carry):
            t0 = pr * 2
            t1 = t0 + 1
            gather_bufs(t1, crB, xrB, nrB, semB)
            wait_bufs(t0, crA, xrA, nrA, semA)
            compute_chunk(t0, crA, xrA, nrA)

            @pl.when(pr < _NCHUNK // 2 - 1)
            def _():
                gather_bufs(t0 + 2, crA, xrA, nrA, semA)

            wait_bufs(t1, crB, xrB, nrB, semB)
            compute_chunk(t1, crB, xrB, nrB)
            return carry

        lax.fori_loop(0, _NCHUNK // 2, pair_body, 0)

        cpo = []
        for j in range(6):
            obase = pl.multiple_of(j * _B + base, _BPW)
            cpo.append(pltpu.async_copy(
                lbuf.at[j], out_hbm.at[pl.ds(obase, _BPW)], semi))
        for cp in cpo:
            cp.wait()

    return sc_logits


def _bce_body(x_ref, o_ref):
    x = x_ref[...]
    pos_rows = _B // 128
    lbl = (lax.broadcasted_iota(jnp.int32, x.shape, 0) < pos_rows
           ).astype(jnp.float32)
    v = jnp.maximum(x, 0.0) - x * lbl + jnp.log(1.0 + jnp.exp(-jnp.abs(x)))
    o_ref[0, 0] = jnp.sum(v) / (6.0 * _B)


def kernel(center, context, negatives, W_in, W_out):
    cen = center.astype(jnp.int32)
    ctx = context.reshape(_B).astype(jnp.int32)
    neg = negatives.reshape(_B * _K).astype(jnp.int32)
    logits = _make_sc_logits()(cen, ctx, neg, W_in, W_out)
    loss = pl.pallas_call(
        _bce_body,
        out_shape=jax.ShapeDtypeStruct((1, 1), jnp.float32),
        out_specs=pl.BlockSpec(memory_space=pltpu.SMEM),
    )(logits.reshape(6 * _B // 128, 128))
    return loss[0, 0]


# speedup vs baseline: 1.6268x; 1.6268x over previous
import functools

import jax
import jax.numpy as jnp
from jax import lax
from jax.experimental import pallas as pl
from jax.experimental.pallas import tpu as pltpu
from jax.experimental.pallas import tpu_sc as plsc

_VOCAB = 100000
_DIM = 64
_B = 16384
_K = 5

_NC = 2
_NS = 16
_NW = _NC * _NS
_BPW = _B // _NW
_S = 128
_NCHUNK = _BPW // _S
_G = _S // 16


@functools.cache
def _make_sc_logits():
    mesh = plsc.VectorSubcoreMesh(core_axis_name="c", subcore_axis_name="s")

    @functools.partial(
        pl.kernel,
        mesh=mesh,
        compiler_params=pltpu.CompilerParams(
            needs_layout_passes=False, use_tc_tiling_on_sc=False),
        out_type=jax.ShapeDtypeStruct((6 * _B,), jnp.float32),
        scratch_types=[
            pltpu.VMEM((_BPW,), jnp.int32),
            pltpu.VMEM((_BPW,), jnp.int32),
            pltpu.VMEM((_K * _BPW,), jnp.int32),
            pltpu.VMEM((_S, _DIM), jnp.float32),
            pltpu.VMEM((_S, _DIM), jnp.float32),
            pltpu.VMEM((_K * _S, _DIM), jnp.float32),
            pltpu.VMEM((_S, _DIM), jnp.float32),
            pltpu.VMEM((_S, _DIM), jnp.float32),
            pltpu.VMEM((_K * _S, _DIM), jnp.float32),
            pltpu.VMEM((6, _BPW), jnp.float32),
            pltpu.SemaphoreType.DMA,
            pltpu.SemaphoreType.DMA,
            pltpu.SemaphoreType.DMA,
        ],
    )
    def sc_logits(cen_hbm, ctx_hbm, neg_hbm, win_hbm, wout_hbm, out_hbm,
                  idxc, idxx, idxn,
                  crA, xrA, nrA, crB, xrB, nrB,
                  lbuf, semi, semA, semB):
        wid = lax.axis_index("s") * _NC + lax.axis_index("c")
        base = wid * _BPW
        iota = lax.iota(jnp.int32, 16)

        cpi = [pltpu.async_copy(cen_hbm.at[pl.ds(base, _BPW)], idxc, semi),
               pltpu.async_copy(ctx_hbm.at[pl.ds(base, _BPW)], idxx, semi),
               pltpu.async_copy(neg_hbm.at[pl.ds(base * _K, _K * _BPW)],
                                idxn, semi)]
        for cp in cpi:
            cp.wait()

        def gather_bufs(t, cr, xr, nr, sem):
            toff = pl.multiple_of(t * _S, _S)
            return [
                pltpu.async_copy(win_hbm.at[idxc.at[pl.ds(toff, _S)]],
                                 cr, sem),
                pltpu.async_copy(wout_hbm.at[idxx.at[pl.ds(toff, _S)]],
                                 xr, sem),
                pltpu.async_copy(wout_hbm.at[idxn.at[pl.ds(toff * _K,
                                                           _K * _S)]],
                                 nr, sem),
            ]

        def wait_bufs(t, cr, xr, nr, sem):
            toff = pl.multiple_of(t * _S, _S)
            pltpu.make_async_copy(win_hbm.at[idxc.at[pl.ds(toff, _S)]],
                                  cr, sem).wait()
            pltpu.make_async_copy(wout_hbm.at[idxx.at[pl.ds(toff, _S)]],
                                  xr, sem).wait()
            pltpu.make_async_copy(wout_hbm.at[idxn.at[pl.ds(toff * _K,
                                                            _K * _S)]],
                                  nr, sem).wait()

        def compute_chunk(t, cr, xr, nr):
            toff = pl.multiple_of(t * _S, _S)

            def g_body(g, carry):
                s0 = pl.multiple_of(g * 16, 16)
                accs = [jnp.zeros((16,), jnp.float32) for _ in range(6)]
                for l in range(16):
                    s = s0 + l
                    lane = iota == l
                    cvs = [cr[s, pl.ds(k * 16, 16)]
                           for k in range(_DIM // 16)]
                    for j in range(6):
                        if j == 0:
                            rvs = [xr[s, pl.ds(k * 16, 16)]
                                   for k in range(_DIM // 16)]
                        else:
                            rvs = [nr[s * _K + (j - 1), pl.ds(k * 16, 16)]
                                   for k in range(_DIM // 16)]
                        p = cvs[0] * rvs[0]
                        for k in range(1, _DIM // 16):
                            p = p + cvs[k] * rvs[k]
                        r = jnp.sum(p)
                        accs[j] = jnp.where(lane, r, accs[j])
                for j in range(6):
                    lbuf[j, pl.ds(toff + s0, 16)] = accs[j]
                return carry

            lax.fori_loop(0, _G, g_body, 0)

        gather_bufs(0, crA, xrA, nrA, semA)

        def pair_body(pr, carry):
            t0 = pr * 2
            t1 = t0 + 1
            gather_bufs(t1, crB, xrB, nrB, semB)
            wait_bufs(t0, crA, xrA, nrA, semA)
            compute_chunk(t0, crA, xrA, nrA)

            @pl.when(pr < _NCHUNK // 2 - 1)
            def _():
                gather_bufs(t0 + 2, crA, xrA, nrA, semA)

            wait_bufs(t1, crB, xrB, nrB, semB)
            compute_chunk(t1, crB, xrB, nrB)
            return carry

        lax.fori_loop(0, _NCHUNK // 2, pair_body, 0)

        cpo = []
        for j in range(6):
            obase = pl.multiple_of(j * _B + base, _BPW)
            cpo.append(pltpu.async_copy(
                lbuf.at[j], out_hbm.at[pl.ds(obase, _BPW)], semi))
        for cp in cpo:
            cp.wait()

    return sc_logits


def _bce_body(x_ref, o_ref):
    x = x_ref[...]
    pos_rows = _B // 128
    lbl = (lax.broadcasted_iota(jnp.int32, x.shape, 0) < pos_rows
           ).astype(jnp.float32)
    v = jnp.maximum(x, 0.0) - x * lbl + jnp.log(1.0 + jnp.exp(-jnp.abs(x)))
    o_ref[0, 0] = jnp.sum(v) / (6.0 * _B)


def kernel(center, context, negatives, W_in, W_out):
    cen = center.astype(jnp.int32)
    ctx = context.reshape(_B).astype(jnp.int32)
    neg = negatives.reshape(_B * _K).astype(jnp.int32)
    logits = _make_sc_logits()(cen, ctx, neg, W_in, W_out)
    loss = pl.pallas_call(
        _bce_body,
        out_shape=jax.ShapeDtypeStruct((1, 1), jnp.float32),
        out_specs=pl.BlockSpec(memory_space=pltpu.SMEM),
    )(logits.reshape(6 * _B // 128, 128))
    return loss[0, 0]


# speedup vs baseline: 1.6363x; 1.0058x over previous
import functools

import jax
import jax.numpy as jnp
from jax import lax
from jax.experimental import pallas as pl
from jax.experimental.pallas import tpu as pltpu
from jax.experimental.pallas import tpu_sc as plsc

_VOCAB = 100000
_DIM = 64
_B = 16384
_K = 5

_NC = 2
_NS = 16
_NW = _NC * _NS
_BPW = _B // _NW
_S = 64
_NCHUNK = _BPW // _S
_G = _S // 16


@functools.cache
def _make_sc_logits():
    mesh = plsc.VectorSubcoreMesh(core_axis_name="c", subcore_axis_name="s")

    @functools.partial(
        pl.kernel,
        mesh=mesh,
        compiler_params=pltpu.CompilerParams(
            needs_layout_passes=False, use_tc_tiling_on_sc=False),
        out_type=jax.ShapeDtypeStruct((6 * _B,), jnp.float32),
        scratch_types=[
            pltpu.VMEM((_BPW,), jnp.int32),
            pltpu.VMEM((_BPW,), jnp.int32),
            pltpu.VMEM((_K * _BPW,), jnp.int32),
            pltpu.VMEM((_S, _DIM), jnp.float32),
            pltpu.VMEM((_S, _DIM), jnp.float32),
            pltpu.VMEM((_K * _S, _DIM), jnp.float32),
            pltpu.VMEM((_S, _DIM), jnp.float32),
            pltpu.VMEM((_S, _DIM), jnp.float32),
            pltpu.VMEM((_K * _S, _DIM), jnp.float32),
            pltpu.VMEM((6, _BPW), jnp.float32),
            pltpu.SemaphoreType.DMA,
            pltpu.SemaphoreType.DMA,
            pltpu.SemaphoreType.DMA,
        ],
    )
    def sc_logits(cen_hbm, ctx_hbm, neg_hbm, win_hbm, wout_hbm, out_hbm,
                  idxc, idxx, idxn,
                  crA, xrA, nrA, crB, xrB, nrB,
                  lbuf, semi, semA, semB):
        wid = lax.axis_index("s") * _NC + lax.axis_index("c")
        base = wid * _BPW
        iota = lax.iota(jnp.int32, 16)

        cpi = [pltpu.async_copy(cen_hbm.at[pl.ds(base, _BPW)], idxc, semi),
               pltpu.async_copy(ctx_hbm.at[pl.ds(base, _BPW)], idxx, semi),
               pltpu.async_copy(neg_hbm.at[pl.ds(base * _K, _K * _BPW)],
                                idxn, semi)]
        for cp in cpi:
            cp.wait()

        def gather_bufs(t, cr, xr, nr, sem):
            toff = pl.multiple_of(t * _S, _S)
            return [
                pltpu.async_copy(win_hbm.at[idxc.at[pl.ds(toff, _S)]],
                                 cr, sem),
                pltpu.async_copy(wout_hbm.at[idxx.at[pl.ds(toff, _S)]],
                                 xr, sem),
                pltpu.async_copy(wout_hbm.at[idxn.at[pl.ds(toff * _K,
                                                           _K * _S)]],
                                 nr, sem),
            ]

        def wait_bufs(t, cr, xr, nr, sem):
            toff = pl.multiple_of(t * _S, _S)
            pltpu.make_async_copy(win_hbm.at[idxc.at[pl.ds(toff, _S)]],
                                  cr, sem).wait()
            pltpu.make_async_copy(wout_hbm.at[idxx.at[pl.ds(toff, _S)]],
                                  xr, sem).wait()
            pltpu.make_async_copy(wout_hbm.at[idxn.at[pl.ds(toff * _K,
                                                            _K * _S)]],
                                  nr, sem).wait()

        def compute_chunk(t, cr, xr, nr):
            toff = pl.multiple_of(t * _S, _S)

            def g_body(g, carry):
                s0 = pl.multiple_of(g * 16, 16)
                accs = [jnp.zeros((16,), jnp.float32) for _ in range(6)]
                for l in range(16):
                    s = s0 + l
                    lane = iota == l
                    cvs = [cr[s, pl.ds(k * 16, 16)]
                           for k in range(_DIM // 16)]
                    for j in range(6):
                        if j == 0:
                            rvs = [xr[s, pl.ds(k * 16, 16)]
                                   for k in range(_DIM // 16)]
                        else:
                            rvs = [nr[s * _K + (j - 1), pl.ds(k * 16, 16)]
                                   for k in range(_DIM // 16)]
                        p = cvs[0] * rvs[0]
                        for k in range(1, _DIM // 16):
                            p = p + cvs[k] * rvs[k]
                        r = jnp.sum(p)
                        accs[j] = jnp.where(lane, r, accs[j])
                for j in range(6):
                    lbuf[j, pl.ds(toff + s0, 16)] = accs[j]
                return carry

            lax.fori_loop(0, _G, g_body, 0)

        gather_bufs(0, crA, xrA, nrA, semA)

        def pair_body(pr, carry):
            t0 = pr * 2
            t1 = t0 + 1
            gather_bufs(t1, crB, xrB, nrB, semB)
            wait_bufs(t0, crA, xrA, nrA, semA)
            compute_chunk(t0, crA, xrA, nrA)

            @pl.when(pr < _NCHUNK // 2 - 1)
            def _():
                gather_bufs(t0 + 2, crA, xrA, nrA, semA)

            wait_bufs(t1, crB, xrB, nrB, semB)
            compute_chunk(t1, crB, xrB, nrB)
            return carry

        lax.fori_loop(0, _NCHUNK // 2, pair_body, 0)

        cpo = []
        for j in range(6):
            obase = pl.multiple_of(j * _B + base, _BPW)
            cpo.append(pltpu.async_copy(
                lbuf.at[j], out_hbm.at[pl.ds(obase, _BPW)], semi))
        for cp in cpo:
            cp.wait()

    return sc_logits


def _bce_body(x_ref, o_ref):
    x = x_ref[...]
    pos_rows = _B // 128
    lbl = (lax.broadcasted_iota(jnp.int32, x.shape, 0) < pos_rows
           ).astype(jnp.float32)
    v = jnp.maximum(x, 0.0) - x * lbl + jnp.log(1.0 + jnp.exp(-jnp.abs(x)))
    o_ref[0, 0] = jnp.sum(v) / (6.0 * _B)


def kernel(center, context, negatives, W_in, W_out):
    cen = center.astype(jnp.int32)
    ctx = context.reshape(_B).astype(jnp.int32)
    neg = negatives.reshape(_B * _K).astype(jnp.int32)
    logits = _make_sc_logits()(cen, ctx, neg, W_in, W_out)
    loss = pl.pallas_call(
        _bce_body,
        out_shape=jax.ShapeDtypeStruct((1, 1), jnp.float32),
        out_specs=pl.BlockSpec(memory_space=pltpu.SMEM),
    )(logits.reshape(6 * _B // 128, 128))
    return loss[0, 0]


# speedup vs baseline: 1.6371x; 1.0005x over previous
import functools

import jax
import jax.numpy as jnp
from jax import lax
from jax.experimental import pallas as pl
from jax.experimental.pallas import tpu as pltpu
from jax.experimental.pallas import tpu_sc as plsc

_VOCAB = 100000
_DIM = 64
_B = 16384
_K = 5

_NC = 2
_NS = 16
_NW = _NC * _NS
_BPW = _B // _NW
_S = 64
_NCHUNK = _BPW // _S
_G = _S // 16


@functools.cache
def _make_sc_logits():
    mesh = plsc.VectorSubcoreMesh(core_axis_name="c", subcore_axis_name="s")

    @functools.partial(
        pl.kernel,
        mesh=mesh,
        compiler_params=pltpu.CompilerParams(
            needs_layout_passes=False, use_tc_tiling_on_sc=False),
        out_type=jax.ShapeDtypeStruct((6 * _B,), jnp.float32),
        scratch_types=[
            pltpu.VMEM((_BPW,), jnp.int32),
            pltpu.VMEM((_BPW,), jnp.int32),
            pltpu.VMEM((_K * _BPW,), jnp.int32),
            pltpu.VMEM((_S, _DIM), jnp.float32),
            pltpu.VMEM((_S, _DIM), jnp.float32),
            pltpu.VMEM((_K * _S, _DIM), jnp.float32),
            pltpu.VMEM((_S, _DIM), jnp.float32),
            pltpu.VMEM((_S, _DIM), jnp.float32),
            pltpu.VMEM((_K * _S, _DIM), jnp.float32),
            pltpu.VMEM((6, _BPW), jnp.float32),
            pltpu.SemaphoreType.DMA,
            pltpu.SemaphoreType.DMA,
            pltpu.SemaphoreType.DMA,
        ],
    )
    def sc_logits(cen_hbm, ctx_hbm, neg_hbm, win_hbm, wout_hbm, out_hbm,
                  idxc, idxx, idxn,
                  crA, xrA, nrA, crB, xrB, nrB,
                  lbuf, semi, semA, semB):
        wid = lax.axis_index("s") * _NC + lax.axis_index("c")
        base = wid * _BPW
        iota = lax.iota(jnp.int32, 16)

        cpi = [pltpu.async_copy(cen_hbm.at[pl.ds(base, _BPW)], idxc, semi),
               pltpu.async_copy(ctx_hbm.at[pl.ds(base, _BPW)], idxx, semi),
               pltpu.async_copy(neg_hbm.at[pl.ds(base * _K, _K * _BPW)],
                                idxn, semi)]
        for cp in cpi:
            cp.wait()

        def gather_bufs(t, cr, xr, nr, sem):
            toff = pl.multiple_of(t * _S, _S)
            return [
                pltpu.async_copy(win_hbm.at[idxc.at[pl.ds(toff, _S)]],
                                 cr, sem),
                pltpu.async_copy(wout_hbm.at[idxx.at[pl.ds(toff, _S)]],
                                 xr, sem),
                pltpu.async_copy(wout_hbm.at[idxn.at[pl.ds(toff * _K,
                                                           _K * _S)]],
                                 nr, sem),
            ]

        def wait_bufs(t, cr, xr, nr, sem):
            toff = pl.multiple_of(t * _S, _S)
            pltpu.make_async_copy(win_hbm.at[idxc.at[pl.ds(toff, _S)]],
                                  cr, sem).wait()
            pltpu.make_async_copy(wout_hbm.at[idxx.at[pl.ds(toff, _S)]],
                                  xr, sem).wait()
            pltpu.make_async_copy(wout_hbm.at[idxn.at[pl.ds(toff * _K,
                                                            _K * _S)]],
                                  nr, sem).wait()

        def compute_chunk(t, cr, xr, nr):
            toff = pl.multiple_of(t * _S, _S)

            @plsc.parallel_loop(0, _G, 1)
            def g_body(g):
                s0 = pl.multiple_of(g * 16, 16)
                accs = [jnp.zeros((16,), jnp.float32) for _ in range(6)]
                for l in range(16):
                    s = s0 + l
                    lane = iota == l
                    cvs = [cr[s, pl.ds(k * 16, 16)]
                           for k in range(_DIM // 16)]
                    for j in range(6):
                        if j == 0:
                            rvs = [xr[s, pl.ds(k * 16, 16)]
                                   for k in range(_DIM // 16)]
                        else:
                            rvs = [nr[s * _K + (j - 1), pl.ds(k * 16, 16)]
                                   for k in range(_DIM // 16)]
                        p = cvs[0] * rvs[0]
                        for k in range(1, _DIM // 16):
                            p = p + cvs[k] * rvs[k]
                        r = jnp.sum(p)
                        accs[j] = jnp.where(lane, r, accs[j])
                for j in range(6):
                    lbuf[j, pl.ds(toff + s0, 16)] = accs[j]

        gather_bufs(0, crA, xrA, nrA, semA)

        def pair_body(pr, carry):
            t0 = pr * 2
            t1 = t0 + 1
            gather_bufs(t1, crB, xrB, nrB, semB)
            wait_bufs(t0, crA, xrA, nrA, semA)
            compute_chunk(t0, crA, xrA, nrA)

            @pl.when(pr < _NCHUNK // 2 - 1)
            def _():
                gather_bufs(t0 + 2, crA, xrA, nrA, semA)

            wait_bufs(t1, crB, xrB, nrB, semB)
            compute_chunk(t1, crB, xrB, nrB)
            return carry

        lax.fori_loop(0, _NCHUNK // 2, pair_body, 0)

        cpo = []
        for j in range(6):
            obase = pl.multiple_of(j * _B + base, _BPW)
            cpo.append(pltpu.async_copy(
                lbuf.at[j], out_hbm.at[pl.ds(obase, _BPW)], semi))
        for cp in cpo:
            cp.wait()

    return sc_logits


def _bce_body(x_ref, o_ref):
    x = x_ref[...]
    pos_rows = _B // 128
    lbl = (lax.broadcasted_iota(jnp.int32, x.shape, 0) < pos_rows
           ).astype(jnp.float32)
    v = jnp.maximum(x, 0.0) - x * lbl + jnp.log(1.0 + jnp.exp(-jnp.abs(x)))
    o_ref[0, 0] = jnp.sum(v) / (6.0 * _B)


def kernel(center, context, negatives, W_in, W_out):
    cen = center.astype(jnp.int32)
    ctx = context.reshape(_B).astype(jnp.int32)
    neg = negatives.reshape(_B * _K).astype(jnp.int32)
    logits = _make_sc_logits()(cen, ctx, neg, W_in, W_out)
    loss = pl.pallas_call(
        _bce_body,
        out_shape=jax.ShapeDtypeStruct((1, 1), jnp.float32),
        out_specs=pl.BlockSpec(memory_space=pltpu.SMEM),
    )(logits.reshape(6 * _B // 128, 128))
    return loss[0, 0]


# speedup vs baseline: 1.6383x; 1.0008x over previous
import functools

import jax
import jax.numpy as jnp
from jax import lax
from jax.experimental import pallas as pl
from jax.experimental.pallas import tpu as pltpu
from jax.experimental.pallas import tpu_sc as plsc

_VOCAB = 100000
_DIM = 64
_B = 16384
_K = 5

_NC = 2
_NS = 16
_NW = _NC * _NS
_BPW = _B // _NW
_S = 64
_NCHUNK = _BPW // _S
_G = _S // 16


@functools.cache
def _make_sc_logits():
    mesh = plsc.VectorSubcoreMesh(core_axis_name="c", subcore_axis_name="s")

    @functools.partial(
        pl.kernel,
        mesh=mesh,
        compiler_params=pltpu.CompilerParams(
            needs_layout_passes=False, use_tc_tiling_on_sc=False),
        out_type=jax.ShapeDtypeStruct((6 * _B,), jnp.float32),
        scratch_types=[
            pltpu.VMEM((_BPW,), jnp.int32),
            pltpu.VMEM((_BPW,), jnp.int32),
            pltpu.VMEM((_K * _BPW,), jnp.int32),
            pltpu.VMEM((_S, _DIM), jnp.float32),
            pltpu.VMEM((_S, _DIM), jnp.float32),
            pltpu.VMEM((_K * _S, _DIM), jnp.float32),
            pltpu.VMEM((_S, _DIM), jnp.float32),
            pltpu.VMEM((_S, _DIM), jnp.float32),
            pltpu.VMEM((_K * _S, _DIM), jnp.float32),
            pltpu.VMEM((6, _BPW), jnp.float32),
            pltpu.SemaphoreType.DMA,
            pltpu.SemaphoreType.DMA,
            pltpu.SemaphoreType.DMA,
        ],
    )
    def sc_logits(cen_hbm, ctx_hbm, neg_hbm, win_hbm, wout_hbm, out_hbm,
                  idxc, idxx, idxn,
                  crA, xrA, nrA, crB, xrB, nrB,
                  lbuf, semi, semA, semB):
        wid = lax.axis_index("s") * _NC + lax.axis_index("c")
        base = wid * _BPW
        iota = lax.iota(jnp.int32, 16)

        cpi = [pltpu.async_copy(cen_hbm.at[pl.ds(base, _BPW)], idxc, semi),
               pltpu.async_copy(ctx_hbm.at[pl.ds(base, _BPW)], idxx, semi),
               pltpu.async_copy(neg_hbm.at[pl.ds(base * _K, _K * _BPW)],
                                idxn, semi)]
        for cp in cpi:
            cp.wait()

        def gather_bufs(t, cr, xr, nr, sem):
            toff = pl.multiple_of(t * _S, _S)
            return [
                pltpu.async_copy(win_hbm.at[idxc.at[pl.ds(toff, _S)]],
                                 cr, sem),
                pltpu.async_copy(wout_hbm.at[idxx.at[pl.ds(toff, _S)]],
                                 xr, sem),
                pltpu.async_copy(wout_hbm.at[idxn.at[pl.ds(toff * _K,
                                                           _K * _S)]],
                                 nr, sem),
            ]

        def wait_bufs(t, cr, xr, nr, sem):
            toff = pl.multiple_of(t * _S, _S)
            pltpu.make_async_copy(win_hbm.at[idxc.at[pl.ds(toff, _S)]],
                                  cr, sem).wait()
            pltpu.make_async_copy(wout_hbm.at[idxx.at[pl.ds(toff, _S)]],
                                  xr, sem).wait()
            pltpu.make_async_copy(wout_hbm.at[idxn.at[pl.ds(toff * _K,
                                                            _K * _S)]],
                                  nr, sem).wait()

        def compute_chunk(t, cr, xr, nr):
            toff = pl.multiple_of(t * _S, _S)

            def g_body(g, carry):
                s0 = pl.multiple_of(g * 16, 16)
                accs = [jnp.zeros((16,), jnp.float32) for _ in range(6)]
                for l in range(16):
                    s = s0 + l
                    lane = iota == l
                    cvs = [cr[s, pl.ds(k * 16, 16)]
                           for k in range(_DIM // 16)]
                    for j in range(6):
                        if j == 0:
                            rvs = [xr[s, pl.ds(k * 16, 16)]
                                   for k in range(_DIM // 16)]
                        else:
                            rvs = [nr[s * _K + (j - 1), pl.ds(k * 16, 16)]
                                   for k in range(_DIM // 16)]
                        p = cvs[0] * rvs[0]
                        for k in range(1, _DIM // 16):
                            p = p + cvs[k] * rvs[k]
                        r = jnp.sum(p)
                        accs[j] = jnp.where(lane, r, accs[j])
                for j in range(6):
                    lbuf[j, pl.ds(toff + s0, 16)] = accs[j]
                return carry

            lax.fori_loop(0, _G, g_body, 0)

        gather_bufs(0, crA, xrA, nrA, semA)

        def pair_body(pr, carry):
            t0 = pr * 2
            t1 = t0 + 1
            gather_bufs(t1, crB, xrB, nrB, semB)
            wait_bufs(t0, crA, xrA, nrA, semA)
            compute_chunk(t0, crA, xrA, nrA)

            @pl.when(pr < _NCHUNK // 2 - 1)
            def _():
                gather_bufs(t0 + 2, crA, xrA, nrA, semA)

            wait_bufs(t1, crB, xrB, nrB, semB)
            compute_chunk(t1, crB, xrB, nrB)
            return carry

        lax.fori_loop(0, _NCHUNK // 2, pair_body, 0)

        cpo = []
        for j in range(6):
            obase = pl.multiple_of(j * _B + base, _BPW)
            cpo.append(pltpu.async_copy(
                lbuf.at[j], out_hbm.at[pl.ds(obase, _BPW)], semi))
        for cp in cpo:
            cp.wait()

    return sc_logits


def _bce_body(x_ref, o_ref):
    x = x_ref[...]
    pos_rows = _B // 128
    lbl = (lax.broadcasted_iota(jnp.int32, x.shape, 0) < pos_rows
           ).astype(jnp.float32)
    v = jnp.maximum(x, 0.0) - x * lbl + jnp.log(1.0 + jnp.exp(-jnp.abs(x)))
    o_ref[0, 0] = jnp.sum(v) / (6.0 * _B)


def kernel(center, context, negatives, W_in, W_out):
    cen = center.astype(jnp.int32)
    ctx = context.reshape(_B).astype(jnp.int32)
    neg = negatives.reshape(_B * _K).astype(jnp.int32)
    logits = _make_sc_logits()(cen, ctx, neg, W_in, W_out)
    loss = pl.pallas_call(
        _bce_body,
        out_shape=jax.ShapeDtypeStruct((1, 1), jnp.float32),
        out_specs=pl.BlockSpec(memory_space=pltpu.SMEM),
    )(logits.reshape(6 * _B // 128, 128))
    return loss[0, 0]
